# baseline (device time: 22873 ns/iter reference)
import jax
import jax.numpy as jnp
from jax import lax
from jax.experimental import pallas as pl
from jax.experimental.pallas import tpu as pltpu

N_DEV = 4
N_TOK = 1024
D_IN = 256
D_OUT = 512
E_LOCAL = 4
CAP = 51.0
ROWS = N_TOK // N_DEV


def kernel(x, router_W, route_idx, expert_W):
    del router_W

    def body(x_ref, idx_ref, w_ref, out_ref,
             keep_ref, send_buf, recv_buf, send_sems, recv_sems):
        my = lax.axis_index("i")

        barrier = pltpu.get_barrier_semaphore()
        for k in range(1, N_DEV):
            pl.semaphore_signal(
                barrier, inc=1,
                device_id=((my + k) % N_DEV,),
                device_id_type=pl.DeviceIdType.MESH,
            )
        pl.semaphore_wait(barrier, N_DEV - 1)

        idx = idx_ref[:, :]
        local_ids = my * E_LOCAL + lax.broadcasted_iota(
            jnp.int32, (N_TOK, E_LOCAL), 1)
        onehot = (idx == local_ids).astype(jnp.float32)
        row = lax.broadcasted_iota(jnp.int32, (N_TOK, N_TOK), 0)
        col = lax.broadcasted_iota(jnp.int32, (N_TOK, N_TOK), 1)
        strict_lower = (col < row).astype(jnp.float32)
        excl = jnp.dot(strict_lower, onehot,
                       preferred_element_type=jnp.float32)
        keep_ref[:, :] = onehot * (excl < CAP).astype(jnp.float32)

        def chunk_for(row_start):
            xp = x_ref[pl.ds(row_start, ROWS), :]
            kp = keep_ref[pl.ds(row_start, ROWS), :]
            acc = jnp.zeros((ROWS, D_OUT), jnp.float32)
            for le in range(E_LOCAL):
                xm = xp * kp[:, le:le + 1]
                acc = acc + jnp.dot(xm, w_ref[le],
                                    preferred_element_type=jnp.float32)
            return acc

        sends = []
        for k in range(1, N_DEV):
            dest = (my + k) % N_DEV
            send_buf[k - 1, :, :] = chunk_for(dest * ROWS)
            rdma = pltpu.make_async_remote_copy(
                src_ref=send_buf.at[k - 1],
                dst_ref=recv_buf.at[N_DEV - 1 - k],
                send_sem=send_sems.at[k - 1],
                recv_sem=recv_sems.at[N_DEV - 1 - k],
                device_id=(dest,),
                device_id_type=pl.DeviceIdType.MESH,
            )
            rdma.start()
            sends.append(rdma)

        out_ref[:, :] = chunk_for(my * ROWS)

        for r in (2, 1, 0):
            recv = pltpu.make_async_remote_copy(
                src_ref=send_buf.at[0],
                dst_ref=recv_buf.at[r],
                send_sem=send_sems.at[0],
                recv_sem=recv_sems.at[r],
                device_id=(my,),
                device_id_type=pl.DeviceIdType.MESH,
            )
            recv.wait_recv()
            out_ref[:, :] += recv_buf[r, :, :]

        for rdma in sends:
            rdma.wait_send()

    return pl.pallas_call(
        body,
        out_shape=jax.ShapeDtypeStruct((ROWS, D_OUT), jnp.float32),
        in_specs=[
            pl.BlockSpec(memory_space=pltpu.VMEM),
            pl.BlockSpec(memory_space=pltpu.VMEM),
            pl.BlockSpec(memory_space=pltpu.VMEM),
        ],
        out_specs=pl.BlockSpec(memory_space=pltpu.VMEM),
        scratch_shapes=[
            pltpu.VMEM((N_TOK, E_LOCAL), jnp.float32),
            pltpu.VMEM((N_DEV - 1, ROWS, D_OUT), jnp.float32),
            pltpu.VMEM((N_DEV - 1, ROWS, D_OUT), jnp.float32),
            pltpu.SemaphoreType.DMA((N_DEV - 1,)),
            pltpu.SemaphoreType.DMA((N_DEV - 1,)),
        ],
        compiler_params=pltpu.CompilerParams(collective_id=0),
    )(x, route_idx, expert_W)
